# baseline (speedup 1.0000x reference)
import functools

import jax
import jax.numpy as jnp
from jax import lax
from jax.experimental import pallas as pl
from jax.experimental.pallas import tpu as pltpu
from jax.experimental.pallas import tpu_sc as plsc

_L = 16
_W = 128


def _sc_gather(user_id, city_id, device_id, E_user, E_city, E_dev):
    B = user_id.shape[0]
    info = plsc.get_sparse_core_info()
    nw = info.num_cores * info.num_subcores
    per_w = B // nw
    n_ch = per_w // _W
    dims = (E_user.shape[1], E_city.shape[1], E_dev.shape[1])

    tabs = (E_user.reshape(-1, _W), E_city.reshape(-1, _W),
            E_dev.reshape(-1, _W))

    mesh = plsc.VectorSubcoreMesh(core_axis_name="c", subcore_axis_name="s")

    @functools.partial(
        pl.kernel,
        mesh=mesh,
        compiler_params=pltpu.CompilerParams(needs_layout_passes=False),
        out_type=tuple(
            jax.ShapeDtypeStruct((nw, per_w * d // _W, _W), jnp.float32)
            for d in dims),
        scratch_types=[
            pltpu.VMEM((per_w,), jnp.int32),
            pltpu.VMEM((per_w,), jnp.int32),
            pltpu.VMEM((per_w,), jnp.int32),
            pltpu.VMEM((per_w,), jnp.int32),
            pltpu.VMEM((per_w,), jnp.int32),
            pltpu.VMEM((per_w,), jnp.int32),
            pltpu.VMEM((_W, _W), jnp.float32),
            pltpu.VMEM((_W, _W), jnp.float32),
            pltpu.VMEM((per_w * dims[0] // _W, _W), jnp.float32),
            pltpu.VMEM((per_w * dims[1] // _W, _W), jnp.float32),
            pltpu.VMEM((per_w * dims[2] // _W, _W), jnp.float32),
            pltpu.SemaphoreType.DMA,
        ],
    )
    def body(uid_h, cid_h, did_h, eu_h, ec_h, ed_h, ou_h, oc_h, od_h,
             i0, i1, i2, p0, p1, p2, w0, w1, b0, b1, b2, sem):
        wid = lax.axis_index("s") * info.num_cores + lax.axis_index("c")
        base = wid * per_w
        idx_refs = (i0, i1, i2)
        prow_refs = (p0, p1, p2)
        wide_refs = (w0, w1)
        out_vrefs = (b0, b1, b2)
        out_hrefs = (ou_h, oc_h, od_h)
        tab_hrefs = (eu_h, ec_h, ed_h)

        for i_ref, src in zip(idx_refs, (uid_h, cid_h, did_h)):
            pltpu.sync_copy(src.at[pl.ds(base, per_w)], i_ref)

        for t in range(3):
            sh = 2 if dims[t] == 32 else 3
            i_ref, p_ref = idx_refs[t], prow_refs[t]

            def prow_step(g, c, i_ref=i_ref, p_ref=p_ref, sh=sh):
                sl = pl.ds(g * _L, _L)
                p_ref[sl] = lax.shift_right_logical(i_ref[sl], sh)
                return c

            lax.fori_loop(0, per_w // _L, prow_step, 0)

        chunks = [(t, j) for t in range(3) for j in range(n_ch)]

        def start(c):
            t, j = chunks[c]
            sl = pl.ds(j * _W, _W)
            return pltpu.async_copy(
                tab_hrefs[t].at[prow_refs[t].at[sl]],
                wide_refs[c % 2].at[pl.ds(0, _W)], sem)

        def extract(c):
            t, j = chunks[c]
            d = dims[t]
            lg = 5 if d == 32 else 4
            msk_r = (1 << (7 - lg)) - 1
            wide = wide_refs[c % 2]
            i_ref = idx_refs[t]
            buf = out_vrefs[t]
            n_g = _W * d // _L
            k_base = j * _W * d
            iota = lax.iota(jnp.int32, _L)

            def grp(g, c2):
                k0 = g * _L
                kv = k0 + iota
                r = lax.shift_right_logical(kv, lg)
                iv = plsc.load_gather(i_ref, [j * _W + r])
                off = lax.shift_left(iv & msk_r, lg)
                col = off + (kv & (d - 1))
                data = plsc.load_gather(wide, [r, col])
                kg = k_base + k0
                buf[lax.shift_right_logical(kg, 7), pl.ds(kg & (_W - 1), _L)] = data
                return c2

            lax.fori_loop(0, n_g, grp, 0)

        cps = {0: start(0)}
        for c in range(len(chunks)):
            if c + 1 < len(chunks):
                cps[c + 1] = start(c + 1)
            cps[c].wait()
            extract(c)

        for t in range(3):
            pltpu.sync_copy(out_vrefs[t], out_hrefs[t].at[wid])

    ou, oc, od = body(user_id, city_id, device_id, *tabs)
    return (ou.reshape(B, dims[0]), oc.reshape(B, dims[1]),
            od.reshape(B, dims[2]))


def _mlp_body(eu_r, ec_r, ed_r, us_r, wd_r, bd_r, w1_r, b1_r, w2_r, b2_r,
              w3_r, b3_r, out_r):
    hp = jax.lax.Precision.HIGHEST
    dense = jnp.dot(us_r[...], wd_r[...], precision=hp,
                    preferred_element_type=jnp.float32) + bd_r[...]
    w1 = w1_r[...]
    h = (jnp.dot(eu_r[...], w1[0:32, :], precision=hp,
                 preferred_element_type=jnp.float32)
         + jnp.dot(ec_r[...], w1[32:48, :], precision=hp,
                   preferred_element_type=jnp.float32)
         + jnp.dot(ed_r[...], w1[48:64, :], precision=hp,
                   preferred_element_type=jnp.float32)
         + jnp.dot(dense, w1[64:96, :], precision=hp,
                   preferred_element_type=jnp.float32)
         + b1_r[...])
    h = jnp.maximum(h, 0.0)
    h = jnp.maximum(jnp.dot(h, w2_r[...], precision=hp,
                            preferred_element_type=jnp.float32) + b2_r[...], 0.0)
    out_r[...] = jnp.dot(h, w3_r[...], precision=hp,
                         preferred_element_type=jnp.float32) + b3_r[...]


def _mlp(eu, ec, ed, user_stats, W_dense, b_dense, W1, b1, W2, b2, W3, b3):
    B = eu.shape[0]
    blk = 2048
    grid = (B // blk,)
    full = lambda shape: pl.BlockSpec(shape, lambda i: (0, 0))
    batched = lambda d: pl.BlockSpec((blk, d), lambda i: (i, 0))
    return pl.pallas_call(
        _mlp_body,
        grid=grid,
        in_specs=[
            batched(eu.shape[1]),
            batched(ec.shape[1]),
            batched(ed.shape[1]),
            batched(user_stats.shape[1]),
            full(W_dense.shape),
            full((1, b_dense.shape[0])),
            full(W1.shape),
            full((1, b1.shape[0])),
            full(W2.shape),
            full((1, b2.shape[0])),
            full(W3.shape),
            full((1, b3.shape[0])),
        ],
        out_specs=batched(W3.shape[1]),
        out_shape=jax.ShapeDtypeStruct((B, W3.shape[1]), jnp.float32),
    )(eu, ec, ed, user_stats, W_dense, b_dense.reshape(1, -1), W1,
      b1.reshape(1, -1), W2, b2.reshape(1, -1), W3, b3.reshape(1, -1))


def kernel(user_id, city_id, device_id, user_stats, E_user, E_city, E_dev,
           W_dense, b_dense, W1, b1, W2, b2, W3, b3):
    eu, ec, ed = _sc_gather(user_id, city_id, device_id, E_user, E_city, E_dev)
    return _mlp(eu, ec, ed, user_stats, W_dense, b_dense, W1, b1, W2, b2,
                W3, b3)
